# baseline (speedup 1.0000x reference)
import functools

import jax
import jax.numpy as jnp
from jax import lax
from jax.experimental import pallas as pl
from jax.experimental.pallas import tpu as pltpu
from jax.experimental.pallas import tpu_sc as plsc

_B = 4096
_F = 26
_D = 16
_N = _B * _F
_NW = 32
_CHUNK = _N // _NW
_TAU = 0.1
_SIG_HALF = 1.0 / (1.0 + 2.718281828459045 ** (-0.5))


def _sc_body(x_hbm, w_hbm, m_hbm, out_hbm, idx_v, rows_v, mask_v, scale_v,
             sem_w, sem_m):
    wid = lax.axis_index("s") * 2 + lax.axis_index("c")
    base = wid * _CHUNK
    pltpu.sync_copy(x_hbm.at[pl.ds(base, _CHUNK)], idx_v)
    cw = pltpu.async_copy(w_hbm.at[idx_v], rows_v, sem_w)
    cm = pltpu.async_copy(m_hbm.at[idx_v], mask_v, sem_m)
    cm.wait()

    inv_tau = jnp.float32(1.0 / _TAU)
    scale_c = jnp.float32(1.0 / _SIG_HALF)

    def scale_body(g, carry):
        m = mask_v[pl.ds(g * 16, 16)]
        s = scale_c / (1.0 + jnp.exp(m * -inv_tau))
        scale_v[pl.ds(g * 16, 16)] = s
        return carry

    lax.fori_loop(0, _CHUNK // 16, scale_body, 0)
    cw.wait()

    def mul_body(g, carry):
        s = scale_v[pl.ds(g * 16, 16)]
        for j in range(16):
            rows_v[g * 16 + j, :] = rows_v[g * 16 + j, :] * s[j]
        return carry

    lax.fori_loop(0, _CHUNK // 16, mul_body, 0)
    pltpu.sync_copy(rows_v, out_hbm.at[pl.ds(base, _CHUNK)])


@jax.jit
def _sc_lookup(x_flat, weight, mask_flat):
    mesh = plsc.VectorSubcoreMesh(core_axis_name="c", subcore_axis_name="s")
    return pl.kernel(
        _sc_body,
        out_type=jax.ShapeDtypeStruct((_N, _D), jnp.float32),
        mesh=mesh,
        scratch_types=[
            pltpu.VMEM((_CHUNK,), jnp.int32),
            pltpu.VMEM((_CHUNK, _D), jnp.float32),
            pltpu.VMEM((_CHUNK,), jnp.float32),
            pltpu.VMEM((_CHUNK,), jnp.float32),
            pltpu.SemaphoreType.DMA,
            pltpu.SemaphoreType.DMA,
        ],
        compiler_params=pltpu.CompilerParams(use_tc_tiling_on_sc=False),
    )(x_flat, weight, mask_flat)


def kernel(x, weight, mask):
    x_flat = x.reshape(-1).astype(jnp.int32)
    mask_flat = mask.reshape(-1)
    out = _sc_lookup(x_flat, weight, mask_flat)
    return out.reshape(_B, _F, _D)
